# baseline (speedup 1.0000x reference)
import functools

import jax
import jax.numpy as jnp
from jax import lax
from jax.experimental import pallas as pl
from jax.experimental.pallas import tpu as pltpu
from jax.experimental.pallas import tpu_sc as plsc

BATCH = 4096
SEQ_LEN = 200
EMBED_DIM = 64
NUM_EMB = 1000000

NC = 2
NS = 16
NW = NC * NS

BBLK = BATCH // NW
NBUF = 8
NGRP = SEQ_LEN // NBUF


def _gather_kernel(xt_hbm, tab_hbm, out_hbm, idx_v, rows_v, *sems):
    gsems, ssems = sems[:NBUF], sems[NBUF:]
    w = lax.axis_index("s") * NC + lax.axis_index("c")
    b0 = w * BBLK

    pltpu.sync_copy(xt_hbm.at[:, pl.ds(b0, BBLK)], idx_v)

    for b in range(NBUF):
        pltpu.async_copy(tab_hbm.at[idx_v.at[b]], rows_v.at[b], gsems[b])

    def outer(g, carry):
        for b in range(NBUF):
            s = g * NBUF + b
            dst = out_hbm.at[pl.ds(b0, BBLK), s]
            pltpu.make_async_copy(
                tab_hbm.at[idx_v.at[s]], rows_v.at[b], gsems[b]).wait()
            pltpu.async_copy(rows_v.at[b], dst, ssems[b])

        @pl.when(g < NGRP - 1)
        def _refill():
            for b in range(NBUF):
                s = g * NBUF + b
                dst = out_hbm.at[pl.ds(b0, BBLK), s]
                pltpu.make_async_copy(rows_v.at[b], dst, ssems[b]).wait()
                pltpu.async_copy(
                    tab_hbm.at[idx_v.at[s + NBUF]], rows_v.at[b], gsems[b])

        return carry

    lax.fori_loop(0, NGRP, outer, 0)

    for b in range(NBUF):
        s = (NGRP - 1) * NBUF + b
        dst = out_hbm.at[pl.ds(b0, BBLK), s]
        pltpu.make_async_copy(rows_v.at[b], dst, ssems[b]).wait()


TBLK = 1024


def _repack_tc_kernel(in_ref, out_ref):
    t = jnp.transpose(in_ref[...], (1, 0))
    t3 = t.reshape(TBLK // 2, 2, EMBED_DIM)
    out_ref[...] = jnp.concatenate([t3[:, 0, :], t3[:, 1, :]], axis=1)


def _repack(tab2):
    grid = (NUM_EMB + TBLK - 1) // TBLK
    return pl.pallas_call(
        _repack_tc_kernel,
        grid=(grid,),
        in_specs=[pl.BlockSpec((EMBED_DIM, TBLK), lambda i: (0, i))],
        out_specs=pl.BlockSpec((TBLK // 2, 128), lambda i: (i, 0)),
        out_shape=jax.ShapeDtypeStruct((NUM_EMB // 2, 128), jnp.float32),
    )(tab2)


def _gather(xt, table):
    mesh = plsc.VectorSubcoreMesh(core_axis_name="c", subcore_axis_name="s")
    run = functools.partial(
        pl.kernel,
        mesh=mesh,
        compiler_params=pltpu.CompilerParams(
            use_tc_tiling_on_sc=False, needs_layout_passes=False),
        out_type=jax.ShapeDtypeStruct((BATCH, SEQ_LEN, EMBED_DIM), jnp.float32),
        scratch_types=[
            pltpu.VMEM((SEQ_LEN, BBLK), jnp.int32),
            pltpu.VMEM((NBUF, BBLK, EMBED_DIM), jnp.float32),
        ] + [pltpu.SemaphoreType.DMA] * (2 * NBUF),
    )(_gather_kernel)
    return run(xt, table)


def kernel(x, table):
    xt = jnp.transpose(x).astype(jnp.int32)
    packed = _repack(jnp.transpose(table))
    tab_lin = packed.reshape(NUM_EMB, EMBED_DIM)
    return _gather(xt, tab_lin)


# speedup vs baseline: 1.2678x; 1.2678x over previous
import functools

import jax
import jax.numpy as jnp
from jax import lax
from jax.experimental import pallas as pl
from jax.experimental.pallas import tpu as pltpu
from jax.experimental.pallas import tpu_sc as plsc

BATCH = 4096
SEQ_LEN = 200
EMBED_DIM = 64

NC = 2
NS = 16
NW = NC * NS

BBLK = BATCH // NW
NBUF = 8
NGRP = SEQ_LEN // NBUF


def _gather_kernel(xt_hbm, tab_hbm, out_hbm, idx_v, rows_v, *sems):
    gsems, ssems = sems[:NBUF], sems[NBUF:]
    w = lax.axis_index("s") * NC + lax.axis_index("c")
    b0 = w * BBLK

    pltpu.sync_copy(xt_hbm.at[:, pl.ds(b0, BBLK)], idx_v)

    def _dst(s):
        return out_hbm.at[pl.ds(w * (SEQ_LEN // 2) * BBLK + (s // 2) * BBLK,
                                BBLK),
                          pl.ds((s % 2) * EMBED_DIM, EMBED_DIM)]

    for b in range(NBUF):
        pltpu.async_copy(tab_hbm.at[idx_v.at[b]], rows_v.at[b], gsems[b])

    def outer(g, carry):
        for b in range(NBUF):
            s = g * NBUF + b
            pltpu.make_async_copy(
                tab_hbm.at[idx_v.at[s]], rows_v.at[b], gsems[b]).wait()
            pltpu.async_copy(rows_v.at[b], _dst(s), ssems[b])

        @pl.when(g < NGRP - 1)
        def _refill():
            for b in range(NBUF):
                s = g * NBUF + b
                pltpu.make_async_copy(rows_v.at[b], _dst(s), ssems[b]).wait()
                pltpu.async_copy(
                    tab_hbm.at[idx_v.at[s + NBUF]], rows_v.at[b], gsems[b])

        return carry

    lax.fori_loop(0, NGRP, outer, 0)

    for b in range(NBUF):
        s = (NGRP - 1) * NBUF + b
        pltpu.make_async_copy(rows_v.at[b], _dst(s), ssems[b]).wait()


def _gather(xt, table):
    mesh = plsc.VectorSubcoreMesh(core_axis_name="c", subcore_axis_name="s")
    run = functools.partial(
        pl.kernel,
        mesh=mesh,
        compiler_params=pltpu.CompilerParams(
            use_tc_tiling_on_sc=False, needs_layout_passes=False),
        out_type=jax.ShapeDtypeStruct((BATCH * SEQ_LEN // 2, 2 * EMBED_DIM),
                                      jnp.float32),
        scratch_types=[
            pltpu.VMEM((SEQ_LEN, BBLK), jnp.int32),
            pltpu.VMEM((NBUF, BBLK, EMBED_DIM), jnp.float32),
        ] + [pltpu.SemaphoreType.DMA] * (2 * NBUF),
    )(_gather_kernel)
    return run(xt, table)


def kernel(x, table):
    xt = jnp.transpose(x).astype(jnp.int32)
    arr = _gather(xt, table)
    out = (arr.reshape(NW, SEQ_LEN // 2, BBLK, 2 * EMBED_DIM)
           .transpose(0, 2, 1, 3)
           .reshape(BATCH, SEQ_LEN, EMBED_DIM))
    return out
